# Initial kernel scaffold; baseline (speedup 1.0000x reference)
import jax
import jax.numpy as jnp
from jax.experimental import pallas as pl


def kernel(x0, edge_index0, x1, edge_index1, g0_Wl0, g0_bl0, g0_Wr0, g0_Wl1, g0_bl1, g0_Wr1, g1_Wl0, g1_bl0, g1_Wr0, g1_Wl1, g1_bl1, g1_Wr1):
    raise NotImplementedError("write your pallas kernel here")


# speedup vs baseline: 3.3466x; 3.3466x over previous
import jax
import jax.numpy as jnp
from jax import lax
from jax.experimental import pallas as pl
from jax.experimental.pallas import tpu as pltpu
from jax.experimental.pallas import tpu_sc as plsc

N = 10000
D = 128
E = 320000

NTILE = 16
B = 128
NB = 160
KC = 16
NCH = NB // KC
PT = NB * B
EP = NTILE * PT
NP = 10112
RPT = NP // NTILE
CROWS = 80


def _sc_agg_body(x_hbm, src_hbm, dst_hbm, zeros_hbm, iota_hbm,
                 agg_out, cnt_out,
                 agg_sh, cnt_sh, src_v, dst_v, rows_v, cnt_v, iota_v, sem):
    c = lax.axis_index("c")
    s = lax.axis_index("s")
    w = c * NTILE + s
    r0 = s * RPT

    pltpu.sync_copy(zeros_hbm, agg_sh.at[pl.ds(r0, RPT)])
    pltpu.sync_copy(zeros_hbm.at[pl.ds(0, B)], cnt_v)
    pltpu.sync_copy(iota_hbm, iota_v)

    @pl.when(s == 0)
    def _():
        pltpu.sync_copy(zeros_hbm.at[pl.ds(0, CROWS)], cnt_sh)

    plsc.subcore_barrier()

    ones16 = jnp.ones((16,), jnp.float32)

    def outer(k, carry):
        pltpu.sync_copy(src_hbm.at[w, pl.ds(k * KC, KC)], src_v)
        pltpu.sync_copy(dst_hbm.at[w, pl.ds(k * KC, KC)], dst_v)

        def body(j, carry2):
            pltpu.async_copy(x_hbm.at[src_v.at[j]], rows_v, sem).wait()
            pltpu.sync_copy(rows_v, agg_sh.at[dst_v.at[j]], add=True)
            for l in range(B // 16):
                d = dst_v[j, pl.ds(l * 16, 16)]
                plsc.addupdate_scatter(
                    cnt_v, [lax.shift_right_logical(d, 7),
                            lax.bitwise_and(d, 127)], ones16)
            return carry2

        lax.fori_loop(0, KC, body, 0)
        return carry

    lax.fori_loop(0, NCH, outer, 0)

    pltpu.sync_copy(cnt_v, cnt_sh.at[iota_v.at[0]], add=True)

    plsc.subcore_barrier()

    pltpu.sync_copy(agg_sh.at[pl.ds(r0, RPT)], agg_out.at[c, pl.ds(r0, RPT)])

    @pl.when(s == 0)
    def _():
        pltpu.sync_copy(cnt_sh, cnt_out.at[c])


@jax.jit
def _sc_agg(x_cat, src, dst):
    zeros = jnp.zeros((RPT, D), jnp.float32)
    iota = jnp.minimum(jnp.arange(128, dtype=jnp.int32),
                       CROWS - 1).reshape(1, 128)
    mesh = plsc.VectorSubcoreMesh(core_axis_name="c", subcore_axis_name="s")
    return pl.kernel(
        _sc_agg_body,
        out_type=(
            jax.ShapeDtypeStruct((2, NP, D), jnp.float32),
            jax.ShapeDtypeStruct((2, CROWS, 128), jnp.float32),
        ),
        mesh=mesh,
        compiler_params=pltpu.CompilerParams(needs_layout_passes=False),
        scratch_types=[
            pltpu.VMEM_SHARED((NP, D), jnp.float32),
            pltpu.VMEM_SHARED((CROWS, 128), jnp.float32),
            pltpu.VMEM((KC, B), jnp.int32),
            pltpu.VMEM((KC, B), jnp.int32),
            pltpu.VMEM((B, D), jnp.float32),
            pltpu.VMEM((B, 128), jnp.float32),
            pltpu.VMEM((1, 128), jnp.int32),
            pltpu.SemaphoreType.DMA,
        ],
    )(x_cat, src, dst, zeros, iota)


def _tc_combine_body(x_ref, agg_ref, cnt_ref, wl_ref, bl_ref, wr_ref, o_ref):
    cb = cnt_ref[...]
    mean = agg_ref[...] / jnp.maximum(cb, 1.0)
    h = lax.dot_general(x_ref[...], wl_ref[...], (((1,), (1,)), ((), ())),
                        preferred_element_type=jnp.float32)
    h = h + lax.dot_general(mean, wr_ref[...], (((1,), (1,)), ((), ())),
                            preferred_element_type=jnp.float32)
    h = h + bl_ref[...]
    o_ref[...] = jnp.where(h > 0.0, h, jnp.exp(h) - 1.0)


def _tc_combine(x, agg, cnt, Wl, bl, Wr):
    BN = 1000
    grid = (N // BN,)
    return pl.pallas_call(
        _tc_combine_body,
        grid=grid,
        in_specs=[
            pl.BlockSpec((BN, D), lambda i: (i, 0)),
            pl.BlockSpec((BN, D), lambda i: (i, 0)),
            pl.BlockSpec((BN, 1), lambda i: (i, 0)),
            pl.BlockSpec((D, D), lambda i: (0, 0)),
            pl.BlockSpec((1, D), lambda i: (0, 0)),
            pl.BlockSpec((D, D), lambda i: (0, 0)),
        ],
        out_specs=pl.BlockSpec((BN, D), lambda i: (i, 0)),
        out_shape=jax.ShapeDtypeStruct((N, D), jnp.float32),
    )(x, agg, cnt, Wl, bl, Wr)


def _prep_edges(edge_index, src_off):
    src = edge_index[0] + src_off
    dst = edge_index[1]
    pad = EP - E
    src = jnp.concatenate([src, jnp.full((pad,), src_off, jnp.int32)])
    dst = jnp.concatenate([dst, jnp.full((pad,), N, jnp.int32)])
    return src.reshape(NTILE, NB, B), dst.reshape(NTILE, NB, B)


def kernel(x0, edge_index0, x1, edge_index1,
           g0_Wl0, g0_bl0, g0_Wr0, g0_Wl1, g0_bl1, g0_Wr1,
           g1_Wl0, g1_bl0, g1_Wr0, g1_Wl1, g1_bl1, g1_Wr1):
    src0, dst0 = _prep_edges(edge_index0, 0)
    src1, dst1 = _prep_edges(edge_index1, N)
    src = jnp.concatenate([src0, src1], axis=0)
    dst = jnp.concatenate([dst0, dst1], axis=0)
    b0l0 = g0_bl0.reshape(1, D)
    b0l1 = g0_bl1.reshape(1, D)
    b1l0 = g1_bl0.reshape(1, D)
    b1l1 = g1_bl1.reshape(1, D)

    x_cat = jnp.concatenate([x0, x1], axis=0)
    agg, cnt = _sc_agg(x_cat, src, dst)
    cnt0 = cnt[0].reshape(CROWS * 128)[:N].reshape(N, 1)
    cnt1 = cnt[1].reshape(CROWS * 128)[:N].reshape(N, 1)
    y0 = _tc_combine(x0, agg[0, :N], cnt0, g0_Wl0, b0l0, g0_Wr0)
    y1 = _tc_combine(x1, agg[1, :N], cnt1, g1_Wl0, b1l0, g1_Wr0)

    y_cat = jnp.concatenate([y0, y1], axis=0)
    agg2, _ = _sc_agg(y_cat, src, dst)
    out0 = _tc_combine(y0, agg2[0, :N], cnt0, g0_Wl1, b0l1, g0_Wr1)
    out1 = _tc_combine(y1, agg2[1, :N], cnt1, g1_Wl1, b1l1, g1_Wr1)
    return jnp.concatenate([out0, out1], axis=0)


# speedup vs baseline: 3.9004x; 1.1655x over previous
import jax
import jax.numpy as jnp
from jax import lax
from jax.experimental import pallas as pl
from jax.experimental.pallas import tpu as pltpu
from jax.experimental.pallas import tpu_sc as plsc

N = 10000
D = 128
E = 320000

NTILE = 16
B = 128
NB = 160
KC = 16
NCH = NB // KC
PT = NB * B
EP = NTILE * PT
NP = 10112
RPT = NP // NTILE
CROWS = 80


def _sc_agg_body(x_hbm, src_hbm, dst_hbm, zeros_hbm,
                 agg_out, cnt_out,
                 agg_sh, src_v, dst_v, rows_a, rows_b, cnt_v,
                 sem_a, sem_b):
    c = lax.axis_index("c")
    s = lax.axis_index("s")
    w = c * NTILE + s
    r0 = s * RPT

    pltpu.sync_copy(zeros_hbm, agg_sh.at[pl.ds(r0, RPT)])
    pltpu.sync_copy(zeros_hbm.at[pl.ds(0, CROWS)], cnt_v)

    plsc.subcore_barrier()

    ones16 = jnp.ones((16,), jnp.float32)

    def count(j):
        for l in range(B // 16):
            d = dst_v[j, pl.ds(l * 16, 16)]
            plsc.addupdate_scatter(
                cnt_v, [lax.shift_right_logical(d, 7),
                        lax.bitwise_and(d, 127)], ones16)

    def outer(k, carry):
        pltpu.sync_copy(src_hbm.at[w, pl.ds(k * KC, KC)], src_v)
        pltpu.sync_copy(dst_hbm.at[w, pl.ds(k * KC, KC)], dst_v)

        pltpu.async_copy(x_hbm.at[src_v.at[0]], rows_a, sem_a)

        def pair(p, carry2):
            j = 2 * p
            pltpu.async_copy(x_hbm.at[src_v.at[j + 1]], rows_b, sem_b)
            pltpu.make_async_copy(x_hbm.at[src_v.at[j]], rows_a, sem_a).wait()
            pltpu.sync_copy(rows_a, agg_sh.at[dst_v.at[j]], add=True)
            count(j)

            @pl.when(j + 2 < KC)
            def _():
                pltpu.async_copy(x_hbm.at[src_v.at[j + 2]], rows_a, sem_a)

            pltpu.make_async_copy(
                x_hbm.at[src_v.at[j + 1]], rows_b, sem_b).wait()
            pltpu.sync_copy(rows_b, agg_sh.at[dst_v.at[j + 1]], add=True)
            count(j + 1)
            return carry2

        lax.fori_loop(0, KC // 2, pair, 0)
        return carry

    lax.fori_loop(0, NCH, outer, 0)

    pltpu.sync_copy(cnt_v, cnt_out.at[c, s])

    plsc.subcore_barrier()

    pltpu.sync_copy(agg_sh.at[pl.ds(r0, RPT)], agg_out.at[c, pl.ds(r0, RPT)])


@jax.jit
def _sc_agg(x_cat, src, dst):
    zeros = jnp.zeros((RPT, D), jnp.float32)
    mesh = plsc.VectorSubcoreMesh(core_axis_name="c", subcore_axis_name="s")
    return pl.kernel(
        _sc_agg_body,
        out_type=(
            jax.ShapeDtypeStruct((2, NP, D), jnp.float32),
            jax.ShapeDtypeStruct((2, NTILE, CROWS, 128), jnp.float32),
        ),
        mesh=mesh,
        compiler_params=pltpu.CompilerParams(needs_layout_passes=False),
        scratch_types=[
            pltpu.VMEM_SHARED((NP, D), jnp.float32),
            pltpu.VMEM((KC, B), jnp.int32),
            pltpu.VMEM((KC, B), jnp.int32),
            pltpu.VMEM((B, D), jnp.float32),
            pltpu.VMEM((B, D), jnp.float32),
            pltpu.VMEM((CROWS, 128), jnp.float32),
            pltpu.SemaphoreType.DMA,
            pltpu.SemaphoreType.DMA,
        ],
    )(x_cat, src, dst, zeros)


def _tc_cntsum_body(cnt_ref, o_ref):
    o_ref[...] = jnp.sum(cnt_ref[...], axis=0, keepdims=True)


def _tc_cntsum(cnt16):
    return pl.pallas_call(
        _tc_cntsum_body,
        in_specs=[pl.BlockSpec((NTILE, CROWS * 128), lambda: (0, 0))],
        out_specs=pl.BlockSpec((1, CROWS * 128), lambda: (0, 0)),
        out_shape=jax.ShapeDtypeStruct((1, CROWS * 128), jnp.float32),
    )(cnt16)


def _tc_combine_body(x_ref, agg_ref, cnt_ref, wl_ref, bl_ref, wr_ref, o_ref):
    cb = cnt_ref[...]
    mean = agg_ref[...] / jnp.maximum(cb, 1.0)
    h = lax.dot_general(x_ref[...], wl_ref[...], (((1,), (1,)), ((), ())),
                        preferred_element_type=jnp.float32)
    h = h + lax.dot_general(mean, wr_ref[...], (((1,), (1,)), ((), ())),
                            preferred_element_type=jnp.float32)
    h = h + bl_ref[...]
    o_ref[...] = jnp.where(h > 0.0, h, jnp.exp(h) - 1.0)


def _tc_combine(x, agg, cnt, Wl, bl, Wr):
    BN = 1000
    grid = (N // BN,)
    return pl.pallas_call(
        _tc_combine_body,
        grid=grid,
        in_specs=[
            pl.BlockSpec((BN, D), lambda i: (i, 0)),
            pl.BlockSpec((BN, D), lambda i: (i, 0)),
            pl.BlockSpec((BN, 1), lambda i: (i, 0)),
            pl.BlockSpec((D, D), lambda i: (0, 0)),
            pl.BlockSpec((1, D), lambda i: (0, 0)),
            pl.BlockSpec((D, D), lambda i: (0, 0)),
        ],
        out_specs=pl.BlockSpec((BN, D), lambda i: (i, 0)),
        out_shape=jax.ShapeDtypeStruct((N, D), jnp.float32),
    )(x, agg, cnt, Wl, bl, Wr)


def _prep_edges(edge_index, src_off):
    src = edge_index[0] + src_off
    dst = edge_index[1]
    pad = EP - E
    src = jnp.concatenate([src, jnp.full((pad,), src_off, jnp.int32)])
    dst = jnp.concatenate([dst, jnp.full((pad,), N, jnp.int32)])
    return src.reshape(NTILE, NB, B), dst.reshape(NTILE, NB, B)


def kernel(x0, edge_index0, x1, edge_index1,
           g0_Wl0, g0_bl0, g0_Wr0, g0_Wl1, g0_bl1, g0_Wr1,
           g1_Wl0, g1_bl0, g1_Wr0, g1_Wl1, g1_bl1, g1_Wr1):
    src0, dst0 = _prep_edges(edge_index0, 0)
    src1, dst1 = _prep_edges(edge_index1, N)
    src = jnp.concatenate([src0, src1], axis=0)
    dst = jnp.concatenate([dst0, dst1], axis=0)
    b0l0 = g0_bl0.reshape(1, D)
    b0l1 = g0_bl1.reshape(1, D)
    b1l0 = g1_bl0.reshape(1, D)
    b1l1 = g1_bl1.reshape(1, D)

    x_cat = jnp.concatenate([x0, x1], axis=0)
    agg, cnt = _sc_agg(x_cat, src, dst)
    cnt0 = _tc_cntsum(cnt[0].reshape(NTILE, CROWS * 128))
    cnt1 = _tc_cntsum(cnt[1].reshape(NTILE, CROWS * 128))
    cnt0 = cnt0.reshape(CROWS * 128)[:N].reshape(N, 1)
    cnt1 = cnt1.reshape(CROWS * 128)[:N].reshape(N, 1)
    y0 = _tc_combine(x0, agg[0, :N], cnt0, g0_Wl0, b0l0, g0_Wr0)
    y1 = _tc_combine(x1, agg[1, :N], cnt1, g1_Wl0, b1l0, g1_Wr0)

    y_cat = jnp.concatenate([y0, y1], axis=0)
    agg2, _ = _sc_agg(y_cat, src, dst)
    out0 = _tc_combine(y0, agg2[0, :N], cnt0, g0_Wl1, b0l1, g0_Wr1)
    out1 = _tc_combine(y1, agg2[1, :N], cnt1, g1_Wl1, b1l1, g1_Wr1)
    return jnp.concatenate([out0, out1], axis=0)


# speedup vs baseline: 3.9022x; 1.0005x over previous
import jax
import jax.numpy as jnp
from jax import lax
from jax.experimental import pallas as pl
from jax.experimental.pallas import tpu as pltpu
from jax.experimental.pallas import tpu_sc as plsc

N = 10000
D = 128
E = 320000

NTILE = 16
B = 128
NB = 160
KC = 16
NCH = NB // KC
PT = NB * B
EP = NTILE * PT
NP = 10112
RPT = NP // NTILE
NSPLIT = 4
SUB = B // NSPLIT
CROWS = 80


def _sc_agg_body(x_hbm, src_hbm, dst_hbm, zeros_hbm,
                 agg_out, cnt_out,
                 agg_sh, src_v, dst_v, rows_a, rows_b, cnt_v,
                 sem_a, sem_b):
    c = lax.axis_index("c")
    s = lax.axis_index("s")
    w = c * NTILE + s
    r0 = s * RPT

    pltpu.sync_copy(zeros_hbm, agg_sh.at[pl.ds(r0, RPT)])
    pltpu.sync_copy(zeros_hbm.at[pl.ds(0, CROWS)], cnt_v)

    plsc.subcore_barrier()

    ones16 = jnp.ones((16,), jnp.float32)

    def count(j):
        for l in range(B // 16):
            d = dst_v[j, pl.ds(l * 16, 16)]
            plsc.addupdate_scatter(
                cnt_v, [lax.shift_right_logical(d, 7),
                        lax.bitwise_and(d, 127)], ones16)

    def outer(k, carry):
        pltpu.sync_copy(src_hbm.at[w, pl.ds(k * KC, KC)], src_v)
        pltpu.sync_copy(dst_hbm.at[w, pl.ds(k * KC, KC)], dst_v)

        def start_gather(buf, j, sem):
            for q in range(NSPLIT):
                pltpu.async_copy(
                    x_hbm.at[src_v.at[j, pl.ds(q * SUB, SUB)]],
                    buf.at[pl.ds(q * SUB, SUB)], sem)

        def wait_gather(buf, j, sem):
            for q in range(NSPLIT):
                pltpu.make_async_copy(
                    x_hbm.at[src_v.at[j, pl.ds(q * SUB, SUB)]],
                    buf.at[pl.ds(q * SUB, SUB)], sem).wait()

        start_gather(rows_a, 0, sem_a)

        def pair(p, carry2):
            j = 2 * p
            start_gather(rows_b, j + 1, sem_b)
            wait_gather(rows_a, j, sem_a)
            pltpu.sync_copy(rows_a, agg_sh.at[dst_v.at[j]], add=True)
            count(j)

            @pl.when(j + 2 < KC)
            def _():
                start_gather(rows_a, j + 2, sem_a)

            wait_gather(rows_b, j + 1, sem_b)
            pltpu.sync_copy(rows_b, agg_sh.at[dst_v.at[j + 1]], add=True)
            count(j + 1)
            return carry2

        lax.fori_loop(0, KC // 2, pair, 0)
        return carry

    lax.fori_loop(0, NCH, outer, 0)

    pltpu.sync_copy(cnt_v, cnt_out.at[c, s])

    plsc.subcore_barrier()

    pltpu.sync_copy(agg_sh.at[pl.ds(r0, RPT)], agg_out.at[c, pl.ds(r0, RPT)])


@jax.jit
def _sc_agg(x_cat, src, dst):
    zeros = jnp.zeros((RPT, D), jnp.float32)
    mesh = plsc.VectorSubcoreMesh(core_axis_name="c", subcore_axis_name="s")
    return pl.kernel(
        _sc_agg_body,
        out_type=(
            jax.ShapeDtypeStruct((2, NP, D), jnp.float32),
            jax.ShapeDtypeStruct((2, NTILE, CROWS, 128), jnp.float32),
        ),
        mesh=mesh,
        compiler_params=pltpu.CompilerParams(needs_layout_passes=False),
        scratch_types=[
            pltpu.VMEM_SHARED((NP, D), jnp.float32),
            pltpu.VMEM((KC, B), jnp.int32),
            pltpu.VMEM((KC, B), jnp.int32),
            pltpu.VMEM((B, D), jnp.float32),
            pltpu.VMEM((B, D), jnp.float32),
            pltpu.VMEM((CROWS, 128), jnp.float32),
            pltpu.SemaphoreType.DMA,
            pltpu.SemaphoreType.DMA,
        ],
    )(x_cat, src, dst, zeros)


def _tc_cntsum_body(cnt_ref, o_ref):
    o_ref[...] = jnp.sum(cnt_ref[...], axis=0, keepdims=True)


def _tc_cntsum(cnt16):
    return pl.pallas_call(
        _tc_cntsum_body,
        in_specs=[pl.BlockSpec((NTILE, CROWS * 128), lambda: (0, 0))],
        out_specs=pl.BlockSpec((1, CROWS * 128), lambda: (0, 0)),
        out_shape=jax.ShapeDtypeStruct((1, CROWS * 128), jnp.float32),
    )(cnt16)


def _tc_combine_body(x_ref, agg_ref, cnt_ref, wl_ref, bl_ref, wr_ref, o_ref):
    cb = cnt_ref[...]
    mean = agg_ref[...] / jnp.maximum(cb, 1.0)
    h = lax.dot_general(x_ref[...], wl_ref[...], (((1,), (1,)), ((), ())),
                        preferred_element_type=jnp.float32)
    h = h + lax.dot_general(mean, wr_ref[...], (((1,), (1,)), ((), ())),
                            preferred_element_type=jnp.float32)
    h = h + bl_ref[...]
    o_ref[...] = jnp.where(h > 0.0, h, jnp.exp(h) - 1.0)


def _tc_combine(x, agg, cnt, Wl, bl, Wr):
    BN = 1000
    grid = (N // BN,)
    return pl.pallas_call(
        _tc_combine_body,
        grid=grid,
        in_specs=[
            pl.BlockSpec((BN, D), lambda i: (i, 0)),
            pl.BlockSpec((BN, D), lambda i: (i, 0)),
            pl.BlockSpec((BN, 1), lambda i: (i, 0)),
            pl.BlockSpec((D, D), lambda i: (0, 0)),
            pl.BlockSpec((1, D), lambda i: (0, 0)),
            pl.BlockSpec((D, D), lambda i: (0, 0)),
        ],
        out_specs=pl.BlockSpec((BN, D), lambda i: (i, 0)),
        out_shape=jax.ShapeDtypeStruct((N, D), jnp.float32),
    )(x, agg, cnt, Wl, bl, Wr)


def _prep_edges(edge_index, src_off):
    src = edge_index[0] + src_off
    dst = edge_index[1]
    pad = EP - E
    src = jnp.concatenate([src, jnp.full((pad,), src_off, jnp.int32)])
    dst = jnp.concatenate([dst, jnp.full((pad,), N, jnp.int32)])
    return src.reshape(NTILE, NB, B), dst.reshape(NTILE, NB, B)


def kernel(x0, edge_index0, x1, edge_index1,
           g0_Wl0, g0_bl0, g0_Wr0, g0_Wl1, g0_bl1, g0_Wr1,
           g1_Wl0, g1_bl0, g1_Wr0, g1_Wl1, g1_bl1, g1_Wr1):
    src0, dst0 = _prep_edges(edge_index0, 0)
    src1, dst1 = _prep_edges(edge_index1, N)
    src = jnp.concatenate([src0, src1], axis=0)
    dst = jnp.concatenate([dst0, dst1], axis=0)
    b0l0 = g0_bl0.reshape(1, D)
    b0l1 = g0_bl1.reshape(1, D)
    b1l0 = g1_bl0.reshape(1, D)
    b1l1 = g1_bl1.reshape(1, D)

    x_cat = jnp.concatenate([x0, x1], axis=0)
    agg, cnt = _sc_agg(x_cat, src, dst)
    cnt0 = _tc_cntsum(cnt[0].reshape(NTILE, CROWS * 128))
    cnt1 = _tc_cntsum(cnt[1].reshape(NTILE, CROWS * 128))
    cnt0 = cnt0.reshape(CROWS * 128)[:N].reshape(N, 1)
    cnt1 = cnt1.reshape(CROWS * 128)[:N].reshape(N, 1)
    y0 = _tc_combine(x0, agg[0, :N], cnt0, g0_Wl0, b0l0, g0_Wr0)
    y1 = _tc_combine(x1, agg[1, :N], cnt1, g1_Wl0, b1l0, g1_Wr0)

    y_cat = jnp.concatenate([y0, y1], axis=0)
    agg2, _ = _sc_agg(y_cat, src, dst)
    out0 = _tc_combine(y0, agg2[0, :N], cnt0, g0_Wl1, b0l1, g0_Wr1)
    out1 = _tc_combine(y1, agg2[1, :N], cnt1, g1_Wl1, b1l1, g1_Wr1)
    return jnp.concatenate([out0, out1], axis=0)


# speedup vs baseline: 5.6938x; 1.4591x over previous
import jax
import jax.numpy as jnp
from jax import lax
from jax.experimental import pallas as pl
from jax.experimental.pallas import tpu as pltpu
from jax.experimental.pallas import tpu_sc as plsc

N = 10000
D = 128
E = 320000

NTILE = 16
B = 128
NB = 160
KC = 16
NCH = NB // KC
PT = NB * B
EP = NTILE * PT
NP = 10112
RPT = NP // NTILE
DH = D // 2
NSPLIT = 2
SUB = B // NSPLIT
CROWS = 80


def _sc_agg_body(x_lo_hbm, x_hi_hbm, src_hbm, dst_hbm, zeros_hbm, zeros_c_hbm,
                 agg_out, cnt_out,
                 x_sp, agg_sh, src_v, dst_v, rows_a, rows_b, cnt_v,
                 sem_a, sem_b):
    c = lax.axis_index("c")
    s = lax.axis_index("s")
    w = c * NTILE + s
    r0 = s * RPT

    pltpu.sync_copy(zeros_c_hbm, cnt_v)

    ones16 = jnp.ones((16,), jnp.float32)

    def count(j):
        for l in range(B // 16):
            d = dst_v[j, pl.ds(l * 16, 16)]
            plsc.addupdate_scatter(
                cnt_v, [lax.shift_right_logical(d, 7),
                        lax.bitwise_and(d, 127)], ones16)

    def half(xh_hbm, h, with_counts):
        pltpu.sync_copy(xh_hbm.at[c, pl.ds(r0, RPT)], x_sp.at[pl.ds(r0, RPT)])
        pltpu.sync_copy(zeros_hbm, agg_sh.at[pl.ds(r0, RPT)])

        plsc.subcore_barrier()

        def outer(k, carry):
            pltpu.sync_copy(src_hbm.at[w, pl.ds(k * KC, KC)], src_v)
            pltpu.sync_copy(dst_hbm.at[w, pl.ds(k * KC, KC)], dst_v)

            def start_gather(buf, j, sem):
                for q in range(NSPLIT):
                    pltpu.async_copy(
                        x_sp.at[src_v.at[j, pl.ds(q * SUB, SUB)]],
                        buf.at[pl.ds(q * SUB, SUB)], sem)

            def wait_gather(buf, j, sem):
                for q in range(NSPLIT):
                    pltpu.make_async_copy(
                        x_sp.at[src_v.at[j, pl.ds(q * SUB, SUB)]],
                        buf.at[pl.ds(q * SUB, SUB)], sem).wait()

            start_gather(rows_a, 0, sem_a)

            def pair(p, carry2):
                j = 2 * p
                start_gather(rows_b, j + 1, sem_b)
                wait_gather(rows_a, j, sem_a)
                pltpu.sync_copy(rows_a, agg_sh.at[dst_v.at[j]], add=True)
                if with_counts:
                    count(j)

                @pl.when(j + 2 < KC)
                def _():
                    start_gather(rows_a, j + 2, sem_a)

                wait_gather(rows_b, j + 1, sem_b)
                pltpu.sync_copy(rows_b, agg_sh.at[dst_v.at[j + 1]], add=True)
                if with_counts:
                    count(j + 1)
                return carry2

            lax.fori_loop(0, KC // 2, pair, 0)
            return carry

        lax.fori_loop(0, NCH, outer, 0)

        plsc.subcore_barrier()

        pltpu.sync_copy(agg_sh.at[pl.ds(r0, RPT)],
                        agg_out.at[h, c, pl.ds(r0, RPT)])

    half(x_lo_hbm, 0, True)
    half(x_hi_hbm, 1, False)

    pltpu.sync_copy(cnt_v, cnt_out.at[c, s])


@jax.jit
def _sc_agg(x_lo, x_hi, src, dst):
    zeros = jnp.zeros((RPT, DH), jnp.float32)
    zeros_c = jnp.zeros((CROWS, 128), jnp.float32)
    mesh = plsc.VectorSubcoreMesh(core_axis_name="c", subcore_axis_name="s")
    return pl.kernel(
        _sc_agg_body,
        out_type=(
            jax.ShapeDtypeStruct((2, 2, NP, DH), jnp.float32),
            jax.ShapeDtypeStruct((2, NTILE, CROWS, 128), jnp.float32),
        ),
        mesh=mesh,
        compiler_params=pltpu.CompilerParams(needs_layout_passes=False),
        scratch_types=[
            pltpu.VMEM_SHARED((NP, DH), jnp.float32),
            pltpu.VMEM_SHARED((NP, DH), jnp.float32),
            pltpu.VMEM((KC, B), jnp.int32),
            pltpu.VMEM((KC, B), jnp.int32),
            pltpu.VMEM((B, DH), jnp.float32),
            pltpu.VMEM((B, DH), jnp.float32),
            pltpu.VMEM((CROWS, 128), jnp.float32),
            pltpu.SemaphoreType.DMA,
            pltpu.SemaphoreType.DMA,
        ],
    )(x_lo, x_hi, src, dst, zeros, zeros_c)


def _tc_cntsum_body(cnt_ref, o_ref):
    o_ref[...] = jnp.sum(cnt_ref[...], axis=0, keepdims=True)


def _tc_cntsum(cnt16):
    return pl.pallas_call(
        _tc_cntsum_body,
        in_specs=[pl.BlockSpec((NTILE, CROWS * 128), lambda: (0, 0))],
        out_specs=pl.BlockSpec((1, CROWS * 128), lambda: (0, 0)),
        out_shape=jax.ShapeDtypeStruct((1, CROWS * 128), jnp.float32),
    )(cnt16)


def _tc_combine_body(x_ref, alo_ref, ahi_ref, cnt_ref, wl_ref, bl_ref,
                     wrlo_ref, wrhi_ref, o_ref):
    inv = 1.0 / jnp.maximum(cnt_ref[...], 1.0)
    h = lax.dot_general(x_ref[...], wl_ref[...], (((1,), (1,)), ((), ())),
                        preferred_element_type=jnp.float32)
    h = h + lax.dot_general(alo_ref[...] * inv, wrlo_ref[...],
                            (((1,), (1,)), ((), ())),
                            preferred_element_type=jnp.float32)
    h = h + lax.dot_general(ahi_ref[...] * inv, wrhi_ref[...],
                            (((1,), (1,)), ((), ())),
                            preferred_element_type=jnp.float32)
    h = h + bl_ref[...]
    o_ref[...] = jnp.where(h > 0.0, h, jnp.exp(h) - 1.0)


def _tc_combine(x, alo, ahi, cnt, Wl, bl, Wr):
    BN = 1000
    grid = (N // BN,)
    return pl.pallas_call(
        _tc_combine_body,
        grid=grid,
        in_specs=[
            pl.BlockSpec((BN, D), lambda i: (i, 0)),
            pl.BlockSpec((BN, DH), lambda i: (i, 0)),
            pl.BlockSpec((BN, DH), lambda i: (i, 0)),
            pl.BlockSpec((BN, 1), lambda i: (i, 0)),
            pl.BlockSpec((D, D), lambda i: (0, 0)),
            pl.BlockSpec((1, D), lambda i: (0, 0)),
            pl.BlockSpec((D, DH), lambda i: (0, 0)),
            pl.BlockSpec((D, DH), lambda i: (0, 0)),
        ],
        out_specs=pl.BlockSpec((BN, D), lambda i: (i, 0)),
        out_shape=jax.ShapeDtypeStruct((N, D), jnp.float32),
    )(x, alo, ahi, cnt, Wl, bl, Wr[:, :DH], Wr[:, DH:])


def _prep_edges(edge_index):
    src = edge_index[0]
    dst = edge_index[1]
    pad = EP - E
    src = jnp.concatenate([src, jnp.zeros((pad,), jnp.int32)])
    dst = jnp.concatenate([dst, jnp.full((pad,), N, jnp.int32)])
    return src.reshape(NTILE, NB, B), dst.reshape(NTILE, NB, B)


def _halves(x0, x1):
    pad = jnp.zeros((NP - N, DH), jnp.float32)
    x_lo = jnp.stack([jnp.concatenate([x0[:, :DH], pad]),
                      jnp.concatenate([x1[:, :DH], pad])])
    x_hi = jnp.stack([jnp.concatenate([x0[:, DH:], pad]),
                      jnp.concatenate([x1[:, DH:], pad])])
    return x_lo, x_hi


def kernel(x0, edge_index0, x1, edge_index1,
           g0_Wl0, g0_bl0, g0_Wr0, g0_Wl1, g0_bl1, g0_Wr1,
           g1_Wl0, g1_bl0, g1_Wr0, g1_Wl1, g1_bl1, g1_Wr1):
    src0, dst0 = _prep_edges(edge_index0)
    src1, dst1 = _prep_edges(edge_index1)
    src = jnp.concatenate([src0, src1], axis=0)
    dst = jnp.concatenate([dst0, dst1], axis=0)
    b0l0 = g0_bl0.reshape(1, D)
    b0l1 = g0_bl1.reshape(1, D)
    b1l0 = g1_bl0.reshape(1, D)
    b1l1 = g1_bl1.reshape(1, D)

    x_lo, x_hi = _halves(x0, x1)
    agg, cnt = _sc_agg(x_lo, x_hi, src, dst)
    cnt0 = _tc_cntsum(cnt[0].reshape(NTILE, CROWS * 128))
    cnt1 = _tc_cntsum(cnt[1].reshape(NTILE, CROWS * 128))
    cnt0 = cnt0.reshape(CROWS * 128)[:N].reshape(N, 1)
    cnt1 = cnt1.reshape(CROWS * 128)[:N].reshape(N, 1)
    y0 = _tc_combine(x0, agg[0, 0, :N], agg[1, 0, :N], cnt0,
                     g0_Wl0, b0l0, g0_Wr0)
    y1 = _tc_combine(x1, agg[0, 1, :N], agg[1, 1, :N], cnt1,
                     g1_Wl0, b1l0, g1_Wr0)

    y_lo, y_hi = _halves(y0, y1)
    agg2, _ = _sc_agg(y_lo, y_hi, src, dst)
    out0 = _tc_combine(y0, agg2[0, 0, :N], agg2[1, 0, :N], cnt0,
                       g0_Wl1, b0l1, g0_Wr1)
    out1 = _tc_combine(y1, agg2[0, 1, :N], agg2[1, 1, :N], cnt1,
                       g1_Wl1, b1l1, g1_Wr1)
    return jnp.concatenate([out0, out1], axis=0)


# speedup vs baseline: 5.7204x; 1.0047x over previous
import jax
import jax.numpy as jnp
from jax import lax
from jax.experimental import pallas as pl
from jax.experimental.pallas import tpu as pltpu
from jax.experimental.pallas import tpu_sc as plsc

N = 10000
D = 128
E = 320000

NTILE = 16
B = 128
NB = 160
KC = 16
NCH = NB // KC
PT = NB * B
EP = NTILE * PT
NP = 10112
RPT = NP // NTILE
DH = D // 2
CROWS = 80


def _sc_agg_body(x_lo_hbm, x_hi_hbm, src_hbm, dst_hbm, zeros_hbm, zeros_c_hbm,
                 agg_out, cnt_out,
                 x_sp, agg_sh, src_v, dst_v, rows_a, rows_b,
                 cnt_v, sem_a, sem_b):
    c = lax.axis_index("c")
    s = lax.axis_index("s")
    w = c * NTILE + s
    r0 = s * RPT

    pltpu.sync_copy(zeros_c_hbm, cnt_v)

    ones16 = jnp.ones((16,), jnp.float32)

    def count(j):
        for l in range(B // 16):
            d = dst_v[j, pl.ds(l * 16, 16)]
            plsc.addupdate_scatter(
                cnt_v, [lax.shift_right_logical(d, 7),
                        lax.bitwise_and(d, 127)], ones16)

    def half(xh_hbm, h, count_parity):
        pltpu.sync_copy(xh_hbm.at[c, pl.ds(r0, RPT)], x_sp.at[pl.ds(r0, RPT)])
        pltpu.sync_copy(zeros_hbm, agg_sh.at[pl.ds(r0, RPT)])

        plsc.subcore_barrier()

        bufs = [(rows_a, sem_a), (rows_b, sem_b)]
        NBUF = len(bufs)

        def start_gather(buf, j, sem):
            pltpu.async_copy(x_sp.at[src_v.at[j]], buf, sem)

        def wait_gather(buf, j, sem):
            pltpu.make_async_copy(x_sp.at[src_v.at[j]], buf, sem).wait()

        def outer(k, carry):
            pltpu.sync_copy(src_hbm.at[w, pl.ds(k * KC, KC)], src_v)
            pltpu.sync_copy(dst_hbm.at[w, pl.ds(k * KC, KC)], dst_v)

            for q in range(NBUF):
                start_gather(bufs[q][0], q, bufs[q][1])
            for j in range(KC):
                buf, sem = bufs[j % NBUF]
                wait_gather(buf, j, sem)
                pltpu.sync_copy(buf, agg_sh.at[dst_v.at[j]], add=True)
                if j + NBUF < KC:
                    start_gather(buf, j + NBUF, sem)
                if j % 2 == count_parity:
                    count(j)
            return carry

        lax.fori_loop(0, NCH, outer, 0)

        plsc.subcore_barrier()

        pltpu.sync_copy(agg_sh.at[pl.ds(r0, RPT)],
                        agg_out.at[h, c, pl.ds(r0, RPT)])

    half(x_lo_hbm, 0, 0)
    half(x_hi_hbm, 1, 1)

    pltpu.sync_copy(cnt_v, cnt_out.at[c, s])


@jax.jit
def _sc_agg(x_lo, x_hi, src, dst):
    zeros = jnp.zeros((RPT, DH), jnp.float32)
    zeros_c = jnp.zeros((CROWS, 128), jnp.float32)
    mesh = plsc.VectorSubcoreMesh(core_axis_name="c", subcore_axis_name="s")
    return pl.kernel(
        _sc_agg_body,
        out_type=(
            jax.ShapeDtypeStruct((2, 2, NP, DH), jnp.float32),
            jax.ShapeDtypeStruct((2, NTILE, CROWS, 128), jnp.float32),
        ),
        mesh=mesh,
        compiler_params=pltpu.CompilerParams(needs_layout_passes=False),
        scratch_types=[
            pltpu.VMEM_SHARED((NP, DH), jnp.float32),
            pltpu.VMEM_SHARED((NP, DH), jnp.float32),
            pltpu.VMEM((KC, B), jnp.int32),
            pltpu.VMEM((KC, B), jnp.int32),
            pltpu.VMEM((B, DH), jnp.float32),
            pltpu.VMEM((B, DH), jnp.float32),
            pltpu.VMEM((CROWS, 128), jnp.float32),
            pltpu.SemaphoreType.DMA,
            pltpu.SemaphoreType.DMA,
        ],
    )(x_lo, x_hi, src, dst, zeros, zeros_c)


def _tc_cntsum_body(cnt_ref, o_ref):
    o_ref[...] = jnp.sum(cnt_ref[...], axis=0, keepdims=True)


def _tc_cntsum(cnt16):
    return pl.pallas_call(
        _tc_cntsum_body,
        in_specs=[pl.BlockSpec((NTILE, CROWS * 128), lambda: (0, 0))],
        out_specs=pl.BlockSpec((1, CROWS * 128), lambda: (0, 0)),
        out_shape=jax.ShapeDtypeStruct((1, CROWS * 128), jnp.float32),
    )(cnt16)


def _tc_combine_body(x_ref, alo_ref, ahi_ref, cnt_ref, wl_ref, bl_ref,
                     wrlo_ref, wrhi_ref, o_ref):
    inv = 1.0 / jnp.maximum(cnt_ref[...], 1.0)
    h = lax.dot_general(x_ref[...], wl_ref[...], (((1,), (1,)), ((), ())),
                        preferred_element_type=jnp.float32)
    h = h + lax.dot_general(alo_ref[...] * inv, wrlo_ref[...],
                            (((1,), (1,)), ((), ())),
                            preferred_element_type=jnp.float32)
    h = h + lax.dot_general(ahi_ref[...] * inv, wrhi_ref[...],
                            (((1,), (1,)), ((), ())),
                            preferred_element_type=jnp.float32)
    h = h + bl_ref[...]
    o_ref[...] = jnp.where(h > 0.0, h, jnp.exp(h) - 1.0)


def _tc_combine(x, alo, ahi, cnt, Wl, bl, Wr):
    BN = 1000
    grid = (N // BN,)
    return pl.pallas_call(
        _tc_combine_body,
        grid=grid,
        in_specs=[
            pl.BlockSpec((BN, D), lambda i: (i, 0)),
            pl.BlockSpec((BN, DH), lambda i: (i, 0)),
            pl.BlockSpec((BN, DH), lambda i: (i, 0)),
            pl.BlockSpec((BN, 1), lambda i: (i, 0)),
            pl.BlockSpec((D, D), lambda i: (0, 0)),
            pl.BlockSpec((1, D), lambda i: (0, 0)),
            pl.BlockSpec((D, DH), lambda i: (0, 0)),
            pl.BlockSpec((D, DH), lambda i: (0, 0)),
        ],
        out_specs=pl.BlockSpec((BN, D), lambda i: (i, 0)),
        out_shape=jax.ShapeDtypeStruct((N, D), jnp.float32),
    )(x, alo, ahi, cnt, Wl, bl, Wr[:, :DH], Wr[:, DH:])


def _prep_edges(edge_index):
    src = edge_index[0]
    dst = edge_index[1]
    pad = EP - E
    src = jnp.concatenate([src, jnp.zeros((pad,), jnp.int32)])
    dst = jnp.concatenate([dst, jnp.full((pad,), N, jnp.int32)])
    return src.reshape(NTILE, NB, B), dst.reshape(NTILE, NB, B)


def _halves(x0, x1):
    pad = jnp.zeros((NP - N, DH), jnp.float32)
    x_lo = jnp.stack([jnp.concatenate([x0[:, :DH], pad]),
                      jnp.concatenate([x1[:, :DH], pad])])
    x_hi = jnp.stack([jnp.concatenate([x0[:, DH:], pad]),
                      jnp.concatenate([x1[:, DH:], pad])])
    return x_lo, x_hi


def kernel(x0, edge_index0, x1, edge_index1,
           g0_Wl0, g0_bl0, g0_Wr0, g0_Wl1, g0_bl1, g0_Wr1,
           g1_Wl0, g1_bl0, g1_Wr0, g1_Wl1, g1_bl1, g1_Wr1):
    src0, dst0 = _prep_edges(edge_index0)
    src1, dst1 = _prep_edges(edge_index1)
    src = jnp.concatenate([src0, src1], axis=0)
    dst = jnp.concatenate([dst0, dst1], axis=0)
    b0l0 = g0_bl0.reshape(1, D)
    b0l1 = g0_bl1.reshape(1, D)
    b1l0 = g1_bl0.reshape(1, D)
    b1l1 = g1_bl1.reshape(1, D)

    x_lo, x_hi = _halves(x0, x1)
    agg, cnt = _sc_agg(x_lo, x_hi, src, dst)
    cnt0 = _tc_cntsum(cnt[0].reshape(NTILE, CROWS * 128))
    cnt1 = _tc_cntsum(cnt[1].reshape(NTILE, CROWS * 128))
    cnt0 = cnt0.reshape(CROWS * 128)[:N].reshape(N, 1)
    cnt1 = cnt1.reshape(CROWS * 128)[:N].reshape(N, 1)
    y0 = _tc_combine(x0, agg[0, 0, :N], agg[1, 0, :N], cnt0,
                     g0_Wl0, b0l0, g0_Wr0)
    y1 = _tc_combine(x1, agg[0, 1, :N], agg[1, 1, :N], cnt1,
                     g1_Wl0, b1l0, g1_Wr0)

    y_lo, y_hi = _halves(y0, y1)
    agg2, _ = _sc_agg(y_lo, y_hi, src, dst)
    out0 = _tc_combine(y0, agg2[0, 0, :N], agg2[1, 0, :N], cnt0,
                       g0_Wl1, b0l1, g0_Wr1)
    out1 = _tc_combine(y1, agg2[0, 1, :N], agg2[1, 1, :N], cnt1,
                       g1_Wl1, b1l1, g1_Wr1)
    return jnp.concatenate([out0, out1], axis=0)


# speedup vs baseline: 5.8590x; 1.0242x over previous
import jax
import jax.numpy as jnp
from jax import lax
from jax.experimental import pallas as pl
from jax.experimental.pallas import tpu as pltpu
from jax.experimental.pallas import tpu_sc as plsc

N = 10000
D = 128
E = 320000

NTILE = 16
B = 128
NB = 160
KC = 16
NCH = NB // KC
PT = NB * B
EP = NTILE * PT
NP = 10112
RPT = NP // NTILE
DH = D // 2
CROWS = 80


def _sc_agg_body(x_lo_hbm, x_hi_hbm, src_hbm, dst_hbm, zeros_hbm, zeros_c_hbm,
                 agg_out, cnt_out,
                 x_sp, agg_sh, src_v, dst_v, rows_a, rows_b,
                 cnt_v, sem_a, sem_b, sem_sa, sem_sb):
    c = lax.axis_index("c")
    s = lax.axis_index("s")
    w = c * NTILE + s
    r0 = s * RPT

    pltpu.sync_copy(zeros_c_hbm, cnt_v)

    ones16 = jnp.ones((16,), jnp.float32)

    def count(j):
        for l in range(B // 16):
            d = dst_v[j, pl.ds(l * 16, 16)]
            plsc.addupdate_scatter(
                cnt_v, [lax.shift_right_logical(d, 7),
                        lax.bitwise_and(d, 127)], ones16)

    def half(xh_hbm, h, count_parity):
        pltpu.sync_copy(xh_hbm.at[c, pl.ds(r0, RPT)], x_sp.at[pl.ds(r0, RPT)])
        pltpu.sync_copy(zeros_hbm, agg_sh.at[pl.ds(r0, RPT)])

        plsc.subcore_barrier()

        bufs = [(rows_a, sem_a, sem_sa), (rows_b, sem_b, sem_sb)]

        def start_gather(buf, j, sem):
            pltpu.async_copy(x_sp.at[src_v.at[j]], buf, sem)

        def wait_gather(buf, j, sem):
            pltpu.make_async_copy(x_sp.at[src_v.at[j]], buf, sem).wait()

        def wait_scatter(buf, sem):
            pltpu.make_async_copy(buf, agg_sh.at[dst_v.at[0]], sem).wait()

        def outer(k, carry):
            pltpu.sync_copy(src_hbm.at[w, pl.ds(k * KC, KC)], src_v)
            pltpu.sync_copy(dst_hbm.at[w, pl.ds(k * KC, KC)], dst_v)

            for q in range(2):
                start_gather(bufs[q][0], q, bufs[q][1])
            for j in range(KC):
                buf, gsem, ssem = bufs[j % 2]
                wait_gather(buf, j, gsem)
                pltpu.async_copy(buf, agg_sh.at[dst_v.at[j]], ssem, add=True)
                if j >= 1:
                    pbuf, pgsem, pssem = bufs[(j - 1) % 2]
                    wait_scatter(pbuf, pssem)
                    if j + 1 < KC:
                        start_gather(pbuf, j + 1, pgsem)
                if j % 2 == count_parity:
                    count(j)
            wait_scatter(bufs[(KC - 1) % 2][0], bufs[(KC - 1) % 2][2])
            return carry

        lax.fori_loop(0, NCH, outer, 0)

        plsc.subcore_barrier()

        pltpu.sync_copy(agg_sh.at[pl.ds(r0, RPT)],
                        agg_out.at[h, c, pl.ds(r0, RPT)])

    half(x_lo_hbm, 0, 0)
    half(x_hi_hbm, 1, 1)

    pltpu.sync_copy(cnt_v, cnt_out.at[c, s])


@jax.jit
def _sc_agg(x_lo, x_hi, src, dst):
    zeros = jnp.zeros((RPT, DH), jnp.float32)
    zeros_c = jnp.zeros((CROWS, 128), jnp.float32)
    mesh = plsc.VectorSubcoreMesh(core_axis_name="c", subcore_axis_name="s")
    return pl.kernel(
        _sc_agg_body,
        out_type=(
            jax.ShapeDtypeStruct((2, 2, NP, DH), jnp.float32),
            jax.ShapeDtypeStruct((2, NTILE, CROWS, 128), jnp.float32),
        ),
        mesh=mesh,
        compiler_params=pltpu.CompilerParams(needs_layout_passes=False),
        scratch_types=[
            pltpu.VMEM_SHARED((NP, DH), jnp.float32),
            pltpu.VMEM_SHARED((NP, DH), jnp.float32),
            pltpu.VMEM((KC, B), jnp.int32),
            pltpu.VMEM((KC, B), jnp.int32),
            pltpu.VMEM((B, DH), jnp.float32),
            pltpu.VMEM((B, DH), jnp.float32),
            pltpu.VMEM((CROWS, 128), jnp.float32),
            pltpu.SemaphoreType.DMA,
            pltpu.SemaphoreType.DMA,
            pltpu.SemaphoreType.DMA,
            pltpu.SemaphoreType.DMA,
        ],
    )(x_lo, x_hi, src, dst, zeros, zeros_c)


def _tc_cntsum_body(cnt_ref, o_ref):
    o_ref[...] = jnp.sum(cnt_ref[...], axis=0, keepdims=True)


def _tc_cntsum(cnt16):
    return pl.pallas_call(
        _tc_cntsum_body,
        in_specs=[pl.BlockSpec((NTILE, CROWS * 128), lambda: (0, 0))],
        out_specs=pl.BlockSpec((1, CROWS * 128), lambda: (0, 0)),
        out_shape=jax.ShapeDtypeStruct((1, CROWS * 128), jnp.float32),
    )(cnt16)


def _tc_combine_body(x_ref, alo_ref, ahi_ref, cnt_ref, wl_ref, bl_ref,
                     wrlo_ref, wrhi_ref, o_ref):
    inv = 1.0 / jnp.maximum(cnt_ref[...], 1.0)
    h = lax.dot_general(x_ref[...], wl_ref[...], (((1,), (1,)), ((), ())),
                        preferred_element_type=jnp.float32)
    h = h + lax.dot_general(alo_ref[...] * inv, wrlo_ref[...],
                            (((1,), (1,)), ((), ())),
                            preferred_element_type=jnp.float32)
    h = h + lax.dot_general(ahi_ref[...] * inv, wrhi_ref[...],
                            (((1,), (1,)), ((), ())),
                            preferred_element_type=jnp.float32)
    h = h + bl_ref[...]
    o_ref[...] = jnp.where(h > 0.0, h, jnp.exp(h) - 1.0)


def _tc_combine(x, alo, ahi, cnt, Wl, bl, Wr):
    BN = 1000
    grid = (N // BN,)
    return pl.pallas_call(
        _tc_combine_body,
        grid=grid,
        in_specs=[
            pl.BlockSpec((BN, D), lambda i: (i, 0)),
            pl.BlockSpec((BN, DH), lambda i: (i, 0)),
            pl.BlockSpec((BN, DH), lambda i: (i, 0)),
            pl.BlockSpec((BN, 1), lambda i: (i, 0)),
            pl.BlockSpec((D, D), lambda i: (0, 0)),
            pl.BlockSpec((1, D), lambda i: (0, 0)),
            pl.BlockSpec((D, DH), lambda i: (0, 0)),
            pl.BlockSpec((D, DH), lambda i: (0, 0)),
        ],
        out_specs=pl.BlockSpec((BN, D), lambda i: (i, 0)),
        out_shape=jax.ShapeDtypeStruct((N, D), jnp.float32),
    )(x, alo, ahi, cnt, Wl, bl, Wr[:, :DH], Wr[:, DH:])


def _prep_edges(edge_index):
    src = edge_index[0]
    dst = edge_index[1]
    pad = EP - E
    src = jnp.concatenate([src, jnp.zeros((pad,), jnp.int32)])
    dst = jnp.concatenate([dst, jnp.full((pad,), N, jnp.int32)])
    return src.reshape(NTILE, NB, B), dst.reshape(NTILE, NB, B)


def _halves(x0, x1):
    pad = jnp.zeros((NP - N, DH), jnp.float32)
    x_lo = jnp.stack([jnp.concatenate([x0[:, :DH], pad]),
                      jnp.concatenate([x1[:, :DH], pad])])
    x_hi = jnp.stack([jnp.concatenate([x0[:, DH:], pad]),
                      jnp.concatenate([x1[:, DH:], pad])])
    return x_lo, x_hi


def kernel(x0, edge_index0, x1, edge_index1,
           g0_Wl0, g0_bl0, g0_Wr0, g0_Wl1, g0_bl1, g0_Wr1,
           g1_Wl0, g1_bl0, g1_Wr0, g1_Wl1, g1_bl1, g1_Wr1):
    src0, dst0 = _prep_edges(edge_index0)
    src1, dst1 = _prep_edges(edge_index1)
    src = jnp.concatenate([src0, src1], axis=0)
    dst = jnp.concatenate([dst0, dst1], axis=0)
    b0l0 = g0_bl0.reshape(1, D)
    b0l1 = g0_bl1.reshape(1, D)
    b1l0 = g1_bl0.reshape(1, D)
    b1l1 = g1_bl1.reshape(1, D)

    x_lo, x_hi = _halves(x0, x1)
    agg, cnt = _sc_agg(x_lo, x_hi, src, dst)
    cnt0 = _tc_cntsum(cnt[0].reshape(NTILE, CROWS * 128))
    cnt1 = _tc_cntsum(cnt[1].reshape(NTILE, CROWS * 128))
    cnt0 = cnt0.reshape(CROWS * 128)[:N].reshape(N, 1)
    cnt1 = cnt1.reshape(CROWS * 128)[:N].reshape(N, 1)
    y0 = _tc_combine(x0, agg[0, 0, :N], agg[1, 0, :N], cnt0,
                     g0_Wl0, b0l0, g0_Wr0)
    y1 = _tc_combine(x1, agg[0, 1, :N], agg[1, 1, :N], cnt1,
                     g1_Wl0, b1l0, g1_Wr0)

    y_lo, y_hi = _halves(y0, y1)
    agg2, _ = _sc_agg(y_lo, y_hi, src, dst)
    out0 = _tc_combine(y0, agg2[0, 0, :N], agg2[1, 0, :N], cnt0,
                       g0_Wl1, b0l1, g0_Wr1)
    out1 = _tc_combine(y1, agg2[0, 1, :N], agg2[1, 1, :N], cnt1,
                       g1_Wl1, b1l1, g1_Wr1)
    return jnp.concatenate([out0, out1], axis=0)
